# baseline (speedup 1.0000x reference)
import functools

import jax
import jax.numpy as jnp
from jax import lax
from jax.experimental import pallas as pl
from jax.experimental.pallas import tpu as pltpu
from jax.experimental.pallas import tpu_sc as plsc

_V, _D, _H, _O = 1000, 300, 100, 50
_B, _L = 4096, 50
_OS = 51
_NW = 32
_BPW = _B // _NW
_NG = _BPW // 16
_VP = 1024
_IDXW = _L * _BPW
_GRP = 8 * _O * 16


def _mlp_body(emb_ref, w1_ref, b1_ref, w2_ref, b2_ref, out_ref):
    x = jnp.dot(emb_ref[...], w1_ref[...],
                preferred_element_type=jnp.float32,
                precision=lax.Precision.HIGHEST)
    x = jnp.maximum(x + b1_ref[...], 0.0)
    y = jnp.dot(x, w2_ref[...],
                preferred_element_type=jnp.float32,
                precision=lax.Precision.HIGHEST)
    y = y + b2_ref[...]
    out_ref[...] = jnp.concatenate(
        [y, jnp.zeros((_V, _OS - _O), jnp.float32)], axis=1)


def _mlp_table(emb, W1, b1, W2, b2):
    return pl.pallas_call(
        _mlp_body,
        out_shape=jax.ShapeDtypeStruct((_V, _OS), jnp.float32),
    )(emb, W1, b1.reshape(1, _H), W2, b2.reshape(1, _O))


def _sc_body(t2_hbm, ia_hbm, ip_hbm, in_hbm, mom_hbm, pres_hbm,
             t2_v, slab_v, ia_v, ip_v, in_v, stage_v, pres_v):
    c = lax.axis_index("c")
    s = lax.axis_index("s")
    wid = s * 2 + c

    pltpu.sync_copy(t2_hbm, t2_v)

    zero16 = jnp.zeros((16,), jnp.float32)
    one16 = jnp.ones((16,), jnp.float32)
    pat = lax.iota(jnp.int32, 16) * _L

    def zero_body(i, _):
        pres_v[pl.ds(i * 16, 16)] = zero16
        return 0
    lax.fori_loop(0, 3 * _VP // 16, zero_body, 0, unroll=4)

    for br, (src, dst, poff) in enumerate(
            ((ia_hbm, ia_v, 0), (ip_hbm, ip_v, _VP), (in_hbm, in_v, 2 * _VP))):
        pltpu.sync_copy(src.at[pl.ds(wid * _IDXW, _IDXW)], slab_v)

        def t_body(g, _):
            gb = g * (16 * _L)

            def tl_body2(l, _):
                v = plsc.load_gather(slab_v, [pat + (gb + l)])
                plsc.store_scatter(pres_v, [v + poff], one16)
                dst[pl.ds(l * _BPW + g * 16, 16)] = v * _OS
                return 0
            lax.fori_loop(0, _L, tl_body2, 0, unroll=2)
            return 0
        lax.fori_loop(0, _NG, t_body, 0)

    def g_body(g, _):
        gbase = g * 16

        def o_body(ot, _):
            ob = ot * 2

            def l_body(l, acc):
                ba = ia_v[pl.ds(l * _BPW + gbase, 16)]
                bp = ip_v[pl.ds(l * _BPW + gbase, 16)]
                bn = in_v[pl.ds(l * _BPW + gbase, 16)]
                out = []
                for t in range(2):
                    (sa, sp, sn, saa, spp, snn, sap, san) = acc[8*t:8*t+8]
                    ta = plsc.load_gather(t2_v, [ba + (ob + t)])
                    tp = plsc.load_gather(t2_v, [bp + (ob + t)])
                    tn = plsc.load_gather(t2_v, [bn + (ob + t)])
                    out.extend(
                        (sa + ta, sp + tp, sn + tn,
                         saa + ta * ta, spp + tp * tp, snn + tn * tn,
                         sap + ta * tp, san + ta * tn))
                return tuple(out)

            accs = lax.fori_loop(0, _L, l_body, (zero16,) * 16, unroll=4)
            for t in range(2):
                for m in range(8):
                    stage_v[pl.ds(m * _O * 16 + (ob + t) * 16, 16)] = \
                        accs[8*t + m]
            return 0
        lax.fori_loop(0, _O // 2, o_body, 0)

        pltpu.sync_copy(
            stage_v, mom_hbm.at[pl.ds((wid * _NG + g) * _GRP, _GRP)])
        return 0
    lax.fori_loop(0, _NG, g_body, 0)

    pltpu.sync_copy(pres_v, pres_hbm.at[pl.ds(wid * 3 * _VP, 3 * _VP)])


def _sc_gather(t2_flat, ia, ip, inn):
    mesh = plsc.VectorSubcoreMesh(core_axis_name="c", subcore_axis_name="s")
    fn = functools.partial(
        pl.kernel,
        out_type=[
            jax.ShapeDtypeStruct((_NW * _NG * _GRP,), jnp.float32),
            jax.ShapeDtypeStruct((_NW * 3 * _VP,), jnp.float32),
        ],
        mesh=mesh,
        compiler_params=pltpu.CompilerParams(needs_layout_passes=False),
        scratch_types=[
            pltpu.VMEM((_V * _OS,), jnp.float32),
            pltpu.VMEM((_IDXW,), jnp.int32),
            pltpu.VMEM((_IDXW,), jnp.int32),
            pltpu.VMEM((_IDXW,), jnp.int32),
            pltpu.VMEM((_IDXW,), jnp.int32),
            pltpu.VMEM((_GRP,), jnp.float32),
            pltpu.VMEM((3 * _VP,), jnp.float32),
        ],
    )(_sc_body)
    return fn(t2_flat, ia, ip, inn)


def _fin_body(mom_ref, pres_ref, t2_ref, out_ref):
    i = pl.program_id(0)

    @pl.when(i == 0)
    def _():
        out_ref[...] = jnp.zeros((1, 1), jnp.float32)

    presm = jnp.max(pres_ref[...], axis=0)
    t2 = t2_ref[...][:, :_O]
    rmin = jnp.min(t2, axis=1)
    rmax = jnp.max(t2, axis=1)
    big = jnp.float32(jnp.inf)
    mns, sps = [], []
    for br in range(3):
        mask = presm[br, :_V] > 0.5
        mn = jnp.min(jnp.where(mask, rmin, big))
        mx = jnp.max(jnp.where(mask, rmax, -big))
        mns.append(mn)
        sps.append(mx - mn)

    m = mom_ref[...]
    sa, sp, sn, saa, spp, snn, sap, san = [m[:, k, :] for k in range(8)]
    mna, mnp_, mnn = mns
    Sa, Sp, Sn = sps
    Lf = jnp.float32(_L)
    sap_h = (sap - mnp_ * sa - mna * sp + Lf * mna * mnp_) / (Sa * Sp)
    san_h = (san - mnn * sa - mna * sn + Lf * mna * mnn) / (Sa * Sn)
    saa_h = (saa - 2.0 * mna * sa + Lf * mna * mna) / (Sa * Sa)
    spp_h = (spp - 2.0 * mnp_ * sp + Lf * mnp_ * mnp_) / (Sp * Sp)
    snn_h = (snn - 2.0 * mnn * sn + Lf * mnn * mnn) / (Sn * Sn)
    eps = jnp.float32(1e-8)
    na = jnp.maximum(jnp.sqrt(saa_h), eps)
    npv = jnp.maximum(jnp.sqrt(spp_h), eps)
    nnv = jnp.maximum(jnp.sqrt(snn_h), eps)
    dp = sap_h / (na * npv)
    dn = san_h / (na * nnv)
    loss = jnp.maximum(dp - dn + 0.5, 0.0)
    out_ref[...] += jnp.sum(loss).reshape(1, 1)


def _finalize(mom3, pres, t2):
    n_chunks = 8
    chunk = mom3.shape[0] // n_chunks
    return pl.pallas_call(
        _fin_body,
        grid=(n_chunks,),
        in_specs=[
            pl.BlockSpec((chunk, 8, _O * 16), lambda i: (i, 0, 0)),
            pl.BlockSpec(pres.shape, lambda i: (0, 0, 0)),
            pl.BlockSpec(t2.shape, lambda i: (0, 0)),
        ],
        out_specs=pl.BlockSpec((1, 1), lambda i: (0, 0)),
        out_shape=jax.ShapeDtypeStruct((1, 1), jnp.float32),
    )(mom3, pres, t2)


def kernel(anchor, positive, negative, emb, W1, b1, W2, b2):
    a = anchor.astype(jnp.int32).reshape(-1)
    p = positive.astype(jnp.int32).reshape(-1)
    n = negative.astype(jnp.int32).reshape(-1)

    t2 = _mlp_table(emb, W1, b1, W2, b2)

    mom, pres = _sc_gather(t2.reshape(-1), a, p, n)
    mom3 = mom.reshape(_NW * _NG, 8, _O * 16)
    out = _finalize(mom3, pres.reshape(_NW, 3, _VP), t2)
    return out[0, 0]


# speedup vs baseline: 1.0280x; 1.0280x over previous
import functools

import jax
import jax.numpy as jnp
from jax import lax
from jax.experimental import pallas as pl
from jax.experimental.pallas import tpu as pltpu
from jax.experimental.pallas import tpu_sc as plsc

_V, _D, _H, _O = 1000, 300, 100, 50
_B, _L = 4096, 50
_NW = 32
_BPW = _B // _NW
_NG = _BPW // 16
_VP = 1024
_IDXW = _L * _BPW
_GRP = 8 * _O * 16


def _mlp_body(emb_ref, w1_ref, b1_ref, w2_ref, b2_ref, out_ref):
    x = jnp.dot(emb_ref[...], w1_ref[...],
                preferred_element_type=jnp.float32,
                precision=lax.Precision.HIGHEST)
    x = jnp.maximum(x + b1_ref[...], 0.0)
    y = jnp.dot(x, w2_ref[...],
                preferred_element_type=jnp.float32,
                precision=lax.Precision.HIGHEST)
    out_ref[...] = y + b2_ref[...]


def _mlp_table(emb, W1, b1, W2, b2):
    return pl.pallas_call(
        _mlp_body,
        out_shape=jax.ShapeDtypeStruct((_V, _O), jnp.float32),
    )(emb, W1, b1.reshape(1, _H), W2, b2.reshape(1, _O))


def _sc_body(t2_hbm, idx_hbm, mom_hbm, pres_hbm,
             t2_v, ia_v, ip_v, in_v, stage_v, pres_v):
    c = lax.axis_index("c")
    s = lax.axis_index("s")
    wid = s * 2 + c

    pltpu.sync_copy(t2_hbm, t2_v)
    pltpu.sync_copy(idx_hbm.at[pl.ds((0 * _NW + wid) * _IDXW, _IDXW)], ia_v)
    pltpu.sync_copy(idx_hbm.at[pl.ds((1 * _NW + wid) * _IDXW, _IDXW)], ip_v)
    pltpu.sync_copy(idx_hbm.at[pl.ds((2 * _NW + wid) * _IDXW, _IDXW)], in_v)

    zero16 = jnp.zeros((16,), jnp.float32)
    one16 = jnp.ones((16,), jnp.float32)

    def zero_body(i, _):
        pres_v[pl.ds(i * 16, 16)] = zero16
        return 0
    lax.fori_loop(0, 3 * _VP // 16, zero_body, 0, unroll=4)

    def pres_body(i, _):
        iav = ia_v[pl.ds(i * 16, 16)]
        ipv = ip_v[pl.ds(i * 16, 16)]
        inv = in_v[pl.ds(i * 16, 16)]
        plsc.store_scatter(pres_v, [iav], one16)
        plsc.store_scatter(pres_v, [ipv + _VP], one16)
        plsc.store_scatter(pres_v, [inv + 2 * _VP], one16)
        ia_v[pl.ds(i * 16, 16)] = iav * _O
        ip_v[pl.ds(i * 16, 16)] = ipv * _O
        in_v[pl.ds(i * 16, 16)] = inv * _O
        return 0
    lax.fori_loop(0, _IDXW // 16, pres_body, 0, unroll=2)

    def g_body(g, _):
        gbase = g * 16

        def o_body(ot, _):
            ob = ot * 2

            def l_body(l, acc):
                ba = ia_v[pl.ds(l * _BPW + gbase, 16)]
                bp = ip_v[pl.ds(l * _BPW + gbase, 16)]
                bn = in_v[pl.ds(l * _BPW + gbase, 16)]
                out = []
                for t in range(2):
                    (sa, sp, sn, saa, spp, snn, sap, san) = acc[8*t:8*t+8]
                    ta = plsc.load_gather(t2_v, [ba + (ob + t)])
                    tp = plsc.load_gather(t2_v, [bp + (ob + t)])
                    tn = plsc.load_gather(t2_v, [bn + (ob + t)])
                    out.extend(
                        (sa + ta, sp + tp, sn + tn,
                         saa + ta * ta, spp + tp * tp, snn + tn * tn,
                         sap + ta * tp, san + ta * tn))
                return tuple(out)

            accs = lax.fori_loop(0, _L, l_body, (zero16,) * 16, unroll=4)
            for t in range(2):
                for m in range(8):
                    stage_v[pl.ds(m * _O * 16 + (ob + t) * 16, 16)] = \
                        accs[8*t + m]
            return 0
        lax.fori_loop(0, _O // 2, o_body, 0)

        pltpu.sync_copy(
            stage_v, mom_hbm.at[pl.ds((wid * _NG + g) * _GRP, _GRP)])
        return 0
    lax.fori_loop(0, _NG, g_body, 0)

    pltpu.sync_copy(pres_v, pres_hbm.at[pl.ds(wid * 3 * _VP, 3 * _VP)])


def _sc_gather(t2_flat, idx_flat):
    mesh = plsc.VectorSubcoreMesh(core_axis_name="c", subcore_axis_name="s")
    fn = functools.partial(
        pl.kernel,
        out_type=[
            jax.ShapeDtypeStruct((_NW * _NG * _GRP,), jnp.float32),
            jax.ShapeDtypeStruct((_NW * 3 * _VP,), jnp.float32),
        ],
        mesh=mesh,
        compiler_params=pltpu.CompilerParams(needs_layout_passes=False),
        scratch_types=[
            pltpu.VMEM((_V * _O,), jnp.float32),
            pltpu.VMEM((_IDXW,), jnp.int32),
            pltpu.VMEM((_IDXW,), jnp.int32),
            pltpu.VMEM((_IDXW,), jnp.int32),
            pltpu.VMEM((_GRP,), jnp.float32),
            pltpu.VMEM((3 * _VP,), jnp.float32),
        ],
    )(_sc_body)
    return fn(t2_flat, idx_flat)


def _fin_body(mom_ref, pres_ref, t2_ref, out_ref):
    i = pl.program_id(0)

    @pl.when(i == 0)
    def _():
        out_ref[...] = jnp.zeros((1, 1), jnp.float32)

    presm = jnp.max(pres_ref[...], axis=0)
    t2 = t2_ref[...]
    rmin = jnp.min(t2, axis=1)
    rmax = jnp.max(t2, axis=1)
    big = jnp.float32(jnp.inf)
    mns, sps = [], []
    for br in range(3):
        mask = presm[br, :_V] > 0.5
        mn = jnp.min(jnp.where(mask, rmin, big))
        mx = jnp.max(jnp.where(mask, rmax, -big))
        mns.append(mn)
        sps.append(mx - mn)

    m = mom_ref[...]
    sa, sp, sn, saa, spp, snn, sap, san = [m[:, k, :] for k in range(8)]
    mna, mnp_, mnn = mns
    Sa, Sp, Sn = sps
    Lf = jnp.float32(_L)
    sap_h = (sap - mnp_ * sa - mna * sp + Lf * mna * mnp_) / (Sa * Sp)
    san_h = (san - mnn * sa - mna * sn + Lf * mna * mnn) / (Sa * Sn)
    saa_h = (saa - 2.0 * mna * sa + Lf * mna * mna) / (Sa * Sa)
    spp_h = (spp - 2.0 * mnp_ * sp + Lf * mnp_ * mnp_) / (Sp * Sp)
    snn_h = (snn - 2.0 * mnn * sn + Lf * mnn * mnn) / (Sn * Sn)
    eps = jnp.float32(1e-8)
    na = jnp.maximum(jnp.sqrt(saa_h), eps)
    npv = jnp.maximum(jnp.sqrt(spp_h), eps)
    nnv = jnp.maximum(jnp.sqrt(snn_h), eps)
    dp = sap_h / (na * npv)
    dn = san_h / (na * nnv)
    loss = jnp.maximum(dp - dn + 0.5, 0.0)
    out_ref[...] += jnp.sum(loss).reshape(1, 1)


def _finalize(mom3, pres, t2):
    n_chunks = 8
    chunk = mom3.shape[0] // n_chunks
    return pl.pallas_call(
        _fin_body,
        grid=(n_chunks,),
        in_specs=[
            pl.BlockSpec((chunk, 8, _O * 16), lambda i: (i, 0, 0)),
            pl.BlockSpec(pres.shape, lambda i: (0, 0, 0)),
            pl.BlockSpec(t2.shape, lambda i: (0, 0)),
        ],
        out_specs=pl.BlockSpec((1, 1), lambda i: (0, 0)),
        out_shape=jax.ShapeDtypeStruct((1, 1), jnp.float32),
    )(mom3, pres, t2)


def kernel(anchor, positive, negative, emb, W1, b1, W2, b2):
    a = anchor.astype(jnp.int32)
    p = positive.astype(jnp.int32)
    n = negative.astype(jnp.int32)

    t2 = _mlp_table(emb, W1, b1, W2, b2)

    idx = jnp.stack([a, p, n])
    idx = idx.transpose(0, 2, 1)
    idx = idx.reshape(3, _L, _NW, _BPW)
    idx = idx.transpose(0, 2, 1, 3)

    mom, pres = _sc_gather(t2.reshape(-1), idx.reshape(-1))
    mom3 = mom.reshape(_NW * _NG, 8, _O * 16)
    out = _finalize(mom3, pres.reshape(_NW, 3, _VP), t2)
    return out[0, 0]
